# baseline (speedup 1.0000x reference)
import functools

import jax
import jax.numpy as jnp
from jax import lax
from jax.experimental import pallas as pl
from jax.experimental.pallas import tpu as pltpu
from jax.experimental.pallas import tpu_sc as plsc

NUM_CORES = 2
NUM_SUBCORES = 16
LANES = 16
NW = NUM_CORES * NUM_SUBCORES

BATCH = 16384
HIDDEN = 64
B_PER_W = BATCH // NW
CHUNK = 128
NCHUNKS = B_PER_W // CHUNK


def _sc_body(users_hbm, items_hbm, uf_hbm, if_hbm, ub_hbm, ib_hbm, out_hbm,
             idx_u, idx_i, uf_b, if_b, ub_b, ib_b, out_buf, sem):
    wid = lax.axis_index("s") * NUM_CORES + lax.axis_index("c")
    base = wid * NCHUNKS

    pltpu.sync_copy(users_hbm.at[pl.ds(base, NCHUNKS)], idx_u)
    pltpu.sync_copy(items_hbm.at[pl.ds(base, NCHUNKS)], idx_i)

    lane = jnp.arange(LANES, dtype=jnp.int32)

    for c in range(NCHUNKS):
        cps = [
            pltpu.async_copy(uf_hbm.at[idx_u.at[c]], uf_b, sem),
            pltpu.async_copy(if_hbm.at[idx_i.at[c]], if_b, sem),
            pltpu.async_copy(ub_hbm.at[idx_u.at[c]], ub_b, sem),
            pltpu.async_copy(ib_hbm.at[idx_i.at[c]], ib_b, sem),
        ]
        for cp in cps:
            cp.wait()

        for g in range(CHUNK // LANES):
            row = g * LANES + lane
            def h_step(h, acc):
                col = jnp.full((LANES,), h, dtype=jnp.int32)
                u = plsc.load_gather(uf_b, [row, col])
                v = plsc.load_gather(if_b, [row, col])
                bu = plsc.load_gather(ub_b, [row, col])
                bi = plsc.load_gather(ib_b, [row, col])
                return acc + u * v + bu + bi
            acc = lax.fori_loop(0, HIDDEN, h_step,
                                jnp.zeros((LANES,), jnp.float32))
            out_buf[pl.ds(c * CHUNK + g * LANES, LANES)] = acc

    pltpu.sync_copy(out_buf, out_hbm.at[pl.ds(wid * B_PER_W, B_PER_W)])


@functools.partial(jax.jit, static_argnames=())
def _run(users2d, items2d, user_factors, item_factors, user_biases,
         item_biases):
    mesh = plsc.VectorSubcoreMesh(
        core_axis_name="c", subcore_axis_name="s",
        num_cores=NUM_CORES, num_subcores=NUM_SUBCORES)
    return pl.kernel(
        _sc_body,
        out_type=jax.ShapeDtypeStruct((BATCH,), jnp.float32),
        mesh=mesh,
        compiler_params=pltpu.CompilerParams(needs_layout_passes=False, use_tc_tiling_on_sc=False),
        scratch_types=[
            pltpu.VMEM((NCHUNKS, CHUNK), jnp.int32),
            pltpu.VMEM((NCHUNKS, CHUNK), jnp.int32),
            pltpu.VMEM((CHUNK, HIDDEN), jnp.float32),
            pltpu.VMEM((CHUNK, HIDDEN), jnp.float32),
            pltpu.VMEM((CHUNK, HIDDEN), jnp.float32),
            pltpu.VMEM((CHUNK, HIDDEN), jnp.float32),
            pltpu.VMEM((B_PER_W,), jnp.float32),
            pltpu.SemaphoreType.DMA,
        ],
    )(users2d, items2d, user_factors, item_factors, user_biases,
      item_biases)


def kernel(users, items, user_factors, item_factors, user_biases,
           item_biases):
    users2d = users.reshape(NW * NCHUNKS, CHUNK)
    items2d = items.reshape(NW * NCHUNKS, CHUNK)
    out = _run(users2d, items2d, user_factors, item_factors, user_biases,
               item_biases)
    return out.reshape(BATCH, 1)
